# baseline (speedup 1.0000x reference)
import functools

import jax
import jax.numpy as jnp
from jax import lax
from jax.experimental import pallas as pl
from jax.experimental.pallas import tpu as pltpu
from jax.experimental.pallas import tpu_sc as plsc

_VOCAB = 1000000
_EMBED = 64
_HIDDEN = 128
_OUT = 50
_SEQ = 200
_BATCH = 4096

_NC = 2
_NS = 16
_L = 16
_KV = _EMBED // _L
_NW = _NC * _NS
_BPW = _BATCH // _NW
_SC = 4
_NCHUNK = _SEQ // _SC
_G = 8
_NG = _BPW // _G


_TW = 8192
_HV = 62 * _TW
_VPAD = 2 * _HV
_NBLK_IN = (_VOCAB + _TW - 1) // _TW


def _tc_relayout(embT):
    def body(a_ref, b_ref, o_ref):
        o_ref[:, 0:_EMBED] = jnp.transpose(a_ref[...], (1, 0)).astype(jnp.bfloat16)
        o_ref[:, _EMBED:2 * _EMBED] = jnp.transpose(b_ref[...], (1, 0)).astype(jnp.bfloat16)

    grid = _HV // _TW
    return pl.pallas_call(
        body,
        grid=(grid,),
        in_specs=[
            pl.BlockSpec((_EMBED, _TW), lambda i: (0, i)),
            pl.BlockSpec((_EMBED, _TW),
                         lambda i: (0, jnp.minimum(i + _HV // _TW,
                                                   _NBLK_IN - 1))),
        ],
        out_specs=pl.BlockSpec((_TW, 2 * _EMBED), lambda i: (i, 0)),
        out_shape=jax.ShapeDtypeStruct((_HV, 2 * _EMBED), jnp.bfloat16),
    )(embT, embT)


def _sc_pooled_sums(x, tab):
    mesh = plsc.VectorSubcoreMesh(
        core_axis_name="c", subcore_axis_name="s",
        num_cores=_NC, num_subcores=_NS)

    @functools.partial(
        pl.kernel,
        out_type=jax.ShapeDtypeStruct((_BATCH, _EMBED), jnp.float32),
        mesh=mesh,
        scratch_types=[
            pltpu.VMEM((_SEQ, _BPW), jnp.int32),
            pltpu.VMEM((2, _SC, _BPW, _EMBED), jnp.bfloat16),
            pltpu.VMEM((_BPW, _EMBED), jnp.float32),
            pltpu.SemaphoreType.DMA,
            pltpu.SemaphoreType.DMA,
        ],
        compiler_params=pltpu.CompilerParams(use_tc_tiling_on_sc=False,
                                             needs_layout_passes=False),
    )
    def body(x_hbm, emb_hbm, out_hbm, idx_v, gbuf, acc_v, sem0, sem1):
        wid = lax.axis_index("s") * _NC + lax.axis_index("c")
        base = wid * _BPW
        pltpu.sync_copy(x_hbm.at[:, pl.ds(base, _BPW)], idx_v)
        sems = (sem0, sem1)

        halfv = jnp.int32(_HV)
        corr = jnp.full((_L,), _VPAD - 1, jnp.int32)
        zero_i = jnp.zeros((_L,), jnp.int32)

        def remap_body(r, carry):
            for k in range(_BPW // _L):
                w = idx_v[r, pl.ds(k * _L, _L)]
                w2 = w + w - jnp.where(w >= halfv, corr, zero_i)
                idx_v[r, pl.ds(k * _L, _L)] = w2
            return carry

        lax.fori_loop(0, _SEQ, remap_body, jnp.int32(0))

        def fire(chunk, slot):
            for s in range(_SC):
                pltpu.async_copy(
                    emb_hbm.at[idx_v.at[chunk * _SC + s]],
                    gbuf.at[slot, s], sems[slot])

        def drain(slot):
            for s in range(_SC):
                pltpu.make_async_copy(
                    emb_hbm.at[idx_v.at[0]], gbuf.at[slot, s], sems[slot]).wait()

        def zero_body(r, carry):
            z = jnp.zeros((_L,), jnp.float32)
            for k in range(_KV):
                acc_v[r, pl.ds(k * _L, _L)] = z
            return carry

        lax.fori_loop(0, _BPW, zero_body, jnp.int32(0))

        def process(slot):
            def group_body(g, carry):
                b0 = g * _G
                accs = [[acc_v[b0 + i, pl.ds(k * _L, _L)] for k in range(_KV)]
                        for i in range(_G)]
                for s in range(_SC):
                    for i in range(_G):
                        c0 = gbuf[slot, s, b0 + i, pl.ds(0, 2 * _L)]
                        c1 = gbuf[slot, s, b0 + i, pl.ds(2 * _L, 2 * _L)]
                        a0, a1 = plsc.unpack(c0, format=plsc.PackFormat.INTERLEAVED)
                        a2, a3 = plsc.unpack(c1, format=plsc.PackFormat.INTERLEAVED)
                        for k, a in enumerate((a0, a1, a2, a3)):
                            accs[i][k] = accs[i][k] + a
                for i in range(_G):
                    for k in range(_KV):
                        acc_v[b0 + i, pl.ds(k * _L, _L)] = accs[i][k]
                return carry

            lax.fori_loop(0, _NG, group_body, jnp.int32(0))

        fire(0, 0)

        def pair_body(p, carry):
            fire(2 * p + 1, 1)
            drain(0)
            process(0)

            @pl.when(2 * p + 2 < _NCHUNK)
            def _():
                fire(2 * p + 2, 0)

            drain(1)
            process(1)
            return carry

        lax.fori_loop(0, _NCHUNK // 2, pair_body, jnp.int32(0))

        iot = lax.iota(jnp.int32, _L)
        col_idx = (2 * iot, 2 * iot + 1, 2 * iot + 32, 2 * iot + 33)

        def reorder_body(r, carry):
            vs = [acc_v[r, pl.ds(k * _L, _L)] for k in range(_KV)]
            rows = jnp.full((_L,), 0, jnp.int32) + r
            for k in range(_KV):
                plsc.store_scatter(acc_v, [rows, col_idx[k]], vs[k])
            return carry

        lax.fori_loop(0, _BPW, reorder_body, jnp.int32(0))
        pltpu.sync_copy(acc_v, out_hbm.at[pl.ds(base, _BPW)])

    return body(x, tab)


def _tc_mlp(pooled_sums, W1, b1, W2, b2):
    def body(p_ref, w1_ref, b1_ref, w2_ref, b2_ref, o_ref):
        p = p_ref[...] * (1.0 / _SEQ)
        h = jnp.dot(p, w1_ref[...], preferred_element_type=jnp.float32) + b1_ref[...]
        z = jnp.dot(h, w2_ref[...], preferred_element_type=jnp.float32) + b2_ref[...]
        z = z - jnp.max(z, axis=-1, keepdims=True)
        e = jnp.exp(z)
        o_ref[...] = e / jnp.sum(e, axis=-1, keepdims=True)

    return pl.pallas_call(
        body,
        out_shape=jax.ShapeDtypeStruct((_BATCH, _OUT), jnp.float32),
    )(pooled_sums, W1, b1.reshape(1, _HIDDEN), W2, b2.reshape(1, _OUT))


def kernel(x, emb, W1, b1, W2, b2):
    tab = _tc_relayout(jnp.transpose(emb))
    tab_lin = tab.reshape(_VPAD, _EMBED)
    pooled_sums = _sc_pooled_sums(x, tab_lin)
    return _tc_mlp(pooled_sums, W1, b1, W2, b2)


# speedup vs baseline: 3.0886x; 3.0886x over previous
import functools

import jax
import jax.numpy as jnp
from jax import lax
from jax.experimental import pallas as pl
from jax.experimental.pallas import tpu as pltpu
from jax.experimental.pallas import tpu_sc as plsc

_VOCAB = 1000000
_EMBED = 64
_HIDDEN = 128
_OUT = 50
_SEQ = 200
_BATCH = 4096

_NC = 2
_NS = 16
_L = 16
_KV = _EMBED // _L
_NW = _NC * _NS
_BPW = _BATCH // _NW
_SC = 4
_NCHUNK = _SEQ // _SC
_G = 8
_NG = _BPW // _G


_TW = 8192
_H4 = 1 << 18
_VPAD = 4 * _H4
_NBLK_IN = (_VOCAB + _TW - 1) // _TW
_QBLK = _H4 // _TW


def _bf16_bits(x):
    u = lax.bitcast_convert_type(x, jnp.uint32)
    return (u + jnp.uint32(0x7FFF) + ((u >> 16) & jnp.uint32(1))) >> 16


def _tc_relayout(embT):
    def body(a_ref, b_ref, c_ref, d_ref, o_ref):
        ws = []
        for ref in (a_ref, b_ref, c_ref, d_ref):
            wa = _bf16_bits(ref[0:32, :])
            wb = _bf16_bits(ref[32:64, :])
            ws.append(lax.bitcast_convert_type(wa | (wb << 16), jnp.float32))
        o_ref[...] = jnp.transpose(jnp.concatenate(ws, axis=0), (1, 0))

    def qmap(q):
        return lambda i: (0, jnp.minimum(i + q * _QBLK, _NBLK_IN - 1))

    return pl.pallas_call(
        body,
        grid=(_QBLK,),
        in_specs=[pl.BlockSpec((_EMBED, _TW), qmap(q)) for q in range(4)],
        out_specs=pl.BlockSpec((_TW, 2 * _EMBED), lambda i: (i, 0)),
        out_shape=jax.ShapeDtypeStruct((_H4, 2 * _EMBED), jnp.float32),
    )(embT, embT, embT, embT)


def _sc_pooled_sums(x, tab_lin):
    mesh = plsc.VectorSubcoreMesh(
        core_axis_name="c", subcore_axis_name="s",
        num_cores=_NC, num_subcores=_NS)

    @functools.partial(
        pl.kernel,
        out_type=jax.ShapeDtypeStruct((_BATCH, _EMBED), jnp.float32),
        mesh=mesh,
        scratch_types=[
            pltpu.VMEM((_SEQ, _BPW), jnp.int32),
            pltpu.VMEM((2, _SC, _BPW, _EMBED // 2), jnp.float32),
            pltpu.VMEM((_BPW, _EMBED), jnp.float32),
            pltpu.SemaphoreType.DMA,
            pltpu.SemaphoreType.DMA,
        ],
        compiler_params=pltpu.CompilerParams(use_tc_tiling_on_sc=False,
                                             needs_layout_passes=False),
    )
    def body(x_hbm, emb_hbm, out_hbm, idx_v, gbuf, acc_v, sem0, sem1):
        wid = lax.axis_index("s") * _NC + lax.axis_index("c")
        base = wid * _BPW
        pltpu.sync_copy(x_hbm.at[:, pl.ds(base, _BPW)], idx_v)
        sems = (sem0, sem1)

        def remap_body(r, carry):
            for k in range(_BPW // _L):
                w = idx_v[r, pl.ds(k * _L, _L)]
                w2 = ((w & jnp.int32(_H4 - 1)) << 2) | (w >> 18)
                idx_v[r, pl.ds(k * _L, _L)] = w2
            return carry

        lax.fori_loop(0, _SEQ, remap_body, jnp.int32(0))

        def fire(chunk, slot):
            for s in range(_SC):
                pltpu.async_copy(
                    emb_hbm.at[idx_v.at[chunk * _SC + s]],
                    gbuf.at[slot, s], sems[slot])

        def drain(slot):
            for s in range(_SC):
                pltpu.make_async_copy(
                    emb_hbm.at[idx_v.at[0]], gbuf.at[slot, s], sems[slot]).wait()

        def zero_body(r, carry):
            z = jnp.zeros((_L,), jnp.float32)
            for k in range(_KV):
                acc_v[r, pl.ds(k * _L, _L)] = z
            return carry

        lax.fori_loop(0, _BPW, zero_body, jnp.int32(0))

        def process(slot):
            def group_body(g, carry):
                b0 = g * _G
                accs = [[acc_v[b0 + i, pl.ds(k * _L, _L)] for k in range(_KV)]
                        for i in range(_G)]
                for s in range(_SC):
                    for i in range(_G):
                        c0 = plsc.bitcast(gbuf[slot, s, b0 + i, pl.ds(0, _L)],
                                          jnp.bfloat16)
                        c1 = plsc.bitcast(gbuf[slot, s, b0 + i, pl.ds(_L, _L)],
                                          jnp.bfloat16)
                        e0, e2 = plsc.unpack(c0, format=plsc.PackFormat.INTERLEAVED)
                        e1, e3 = plsc.unpack(c1, format=plsc.PackFormat.INTERLEAVED)
                        for k, a in enumerate((e0, e1, e2, e3)):
                            accs[i][k] = accs[i][k] + a
                for i in range(_G):
                    for k in range(_KV):
                        acc_v[b0 + i, pl.ds(k * _L, _L)] = accs[i][k]
                return carry

            lax.fori_loop(0, _NG, group_body, jnp.int32(0))

        fire(0, 0)

        def pair_body(p, carry):
            fire(2 * p + 1, 1)
            drain(0)
            process(0)

            @pl.when(2 * p + 2 < _NCHUNK)
            def _():
                fire(2 * p + 2, 0)

            drain(1)
            process(1)
            return carry

        lax.fori_loop(0, _NCHUNK // 2, pair_body, jnp.int32(0))
        pltpu.sync_copy(acc_v, out_hbm.at[pl.ds(base, _BPW)])

    return body(x, tab_lin)


def _tc_mlp(pooled_sums, W1, b1, W2, b2):
    def body(p_ref, w1_ref, b1_ref, w2_ref, b2_ref, o_ref):
        p = p_ref[...] * (1.0 / _SEQ)
        h = jnp.dot(p, w1_ref[...], preferred_element_type=jnp.float32) + b1_ref[...]
        z = jnp.dot(h, w2_ref[...], preferred_element_type=jnp.float32) + b2_ref[...]
        z = z - jnp.max(z, axis=-1, keepdims=True)
        e = jnp.exp(z)
        o_ref[...] = e / jnp.sum(e, axis=-1, keepdims=True)

    return pl.pallas_call(
        body,
        out_shape=jax.ShapeDtypeStruct((_BATCH, _OUT), jnp.float32),
    )(pooled_sums, W1, b1.reshape(1, _HIDDEN), W2, b2.reshape(1, _OUT))


def kernel(x, emb, W1, b1, W2, b2):
    tab = _tc_relayout(jnp.transpose(emb))
    pooled_sums = _sc_pooled_sums(x, tab.reshape(_VPAD, _EMBED // 2))
    return _tc_mlp(pooled_sums, W1, b1, W2, b2)


# speedup vs baseline: 3.2414x; 1.0495x over previous
import functools

import jax
import jax.numpy as jnp
from jax import lax
from jax.experimental import pallas as pl
from jax.experimental.pallas import tpu as pltpu
from jax.experimental.pallas import tpu_sc as plsc

_VOCAB = 1000000
_EMBED = 64
_HIDDEN = 128
_OUT = 50
_SEQ = 200
_BATCH = 4096

_NC = 2
_NS = 16
_L = 16
_KV = _EMBED // _L
_NW = _NC * _NS
_BPW = _BATCH // _NW
_SC = 8
_NCHUNK = _SEQ // _SC
_G = 8
_NG = _BPW // _G


_TW = 8192
_H4 = 1 << 18
_VPAD = 4 * _H4
_NBLK_IN = (_VOCAB + _TW - 1) // _TW
_QBLK = _H4 // _TW


def _bf16_bits(x):
    u = lax.bitcast_convert_type(x, jnp.uint32)
    return (u + jnp.uint32(0x7FFF) + ((u >> 16) & jnp.uint32(1))) >> 16


def _tc_relayout(embT):
    def body(a_ref, b_ref, c_ref, d_ref, o_ref):
        ws = []
        for ref in (a_ref, b_ref, c_ref, d_ref):
            wa = _bf16_bits(ref[0:32, :])
            wb = _bf16_bits(ref[32:64, :])
            ws.append(lax.bitcast_convert_type(wa | (wb << 16), jnp.float32))
        o_ref[...] = jnp.transpose(jnp.concatenate(ws, axis=0), (1, 0))

    def qmap(q):
        return lambda i: (0, jnp.minimum(i + q * _QBLK, _NBLK_IN - 1))

    return pl.pallas_call(
        body,
        grid=(_QBLK,),
        in_specs=[pl.BlockSpec((_EMBED, _TW), qmap(q)) for q in range(4)],
        out_specs=pl.BlockSpec((_TW, 2 * _EMBED), lambda i: (i, 0)),
        out_shape=jax.ShapeDtypeStruct((_H4, 2 * _EMBED), jnp.float32),
    )(embT, embT, embT, embT)


def _sc_pooled_sums(x, tab_lin):
    mesh = plsc.VectorSubcoreMesh(
        core_axis_name="c", subcore_axis_name="s",
        num_cores=_NC, num_subcores=_NS)

    @functools.partial(
        pl.kernel,
        out_type=jax.ShapeDtypeStruct((_BATCH, _EMBED), jnp.float32),
        mesh=mesh,
        scratch_types=[
            pltpu.VMEM((_SEQ, _BPW), jnp.int32),
            pltpu.VMEM((2, _SC, _BPW, _EMBED // 2), jnp.float32),
            pltpu.VMEM((_BPW, _EMBED), jnp.float32),
            pltpu.SemaphoreType.DMA,
            pltpu.SemaphoreType.DMA,
        ],
        compiler_params=pltpu.CompilerParams(use_tc_tiling_on_sc=False,
                                             needs_layout_passes=False),
    )
    def body(x_hbm, emb_hbm, out_hbm, idx_v, gbuf, acc_v, sem0, sem1):
        wid = lax.axis_index("s") * _NC + lax.axis_index("c")
        base = wid * _BPW
        pltpu.sync_copy(x_hbm.at[:, pl.ds(base, _BPW)], idx_v)
        sems = (sem0, sem1)

        def remap_body(r, carry):
            for k in range(_BPW // _L):
                w = idx_v[r, pl.ds(k * _L, _L)]
                w2 = ((w & jnp.int32(_H4 - 1)) << 2) | (w >> 18)
                idx_v[r, pl.ds(k * _L, _L)] = w2
            return carry

        lax.fori_loop(0, _SEQ, remap_body, jnp.int32(0))

        def fire(chunk, slot):
            for s in range(_SC):
                pltpu.async_copy(
                    emb_hbm.at[idx_v.at[chunk * _SC + s]],
                    gbuf.at[slot, s], sems[slot])

        def drain(slot):
            for s in range(_SC):
                pltpu.make_async_copy(
                    emb_hbm.at[idx_v.at[0]], gbuf.at[slot, s], sems[slot]).wait()

        def zero_body(r, carry):
            z = jnp.zeros((_L,), jnp.float32)
            for k in range(_KV):
                acc_v[r, pl.ds(k * _L, _L)] = z
            return carry

        lax.fori_loop(0, _BPW, zero_body, jnp.int32(0))

        def process(slot):
            def group_body(g, carry):
                b0 = g * _G
                accs = [[acc_v[b0 + i, pl.ds(k * _L, _L)] for k in range(_KV)]
                        for i in range(_G)]
                for s in range(_SC):
                    for i in range(_G):
                        c0 = plsc.bitcast(gbuf[slot, s, b0 + i, pl.ds(0, _L)],
                                          jnp.bfloat16)
                        c1 = plsc.bitcast(gbuf[slot, s, b0 + i, pl.ds(_L, _L)],
                                          jnp.bfloat16)
                        e0, e2 = plsc.unpack(c0, format=plsc.PackFormat.INTERLEAVED)
                        e1, e3 = plsc.unpack(c1, format=plsc.PackFormat.INTERLEAVED)
                        for k, a in enumerate((e0, e1, e2, e3)):
                            accs[i][k] = accs[i][k] + a
                for i in range(_G):
                    for k in range(_KV):
                        acc_v[b0 + i, pl.ds(k * _L, _L)] = accs[i][k]
                return carry

            lax.fori_loop(0, _NG, group_body, jnp.int32(0))

        fire(0, 0)

        def pair_body(p, carry):
            fire(2 * p + 1, 1)
            drain(0)
            process(0)

            @pl.when(2 * p + 2 < _NCHUNK)
            def _():
                fire(2 * p + 2, 0)

            drain(1)
            process(1)
            return carry

        lax.fori_loop(0, _NCHUNK // 2, pair_body, jnp.int32(0))
        if _NCHUNK % 2:
            drain(0)
            process(0)
        pltpu.sync_copy(acc_v, out_hbm.at[pl.ds(base, _BPW)])

    return body(x, tab_lin)


def _tc_mlp(pooled_sums, W1, b1, W2, b2):
    def body(p_ref, w1_ref, b1_ref, w2_ref, b2_ref, o_ref):
        p = p_ref[...] * (1.0 / _SEQ)
        h = jnp.dot(p, w1_ref[...], preferred_element_type=jnp.float32) + b1_ref[...]
        z = jnp.dot(h, w2_ref[...], preferred_element_type=jnp.float32) + b2_ref[...]
        z = z - jnp.max(z, axis=-1, keepdims=True)
        e = jnp.exp(z)
        o_ref[...] = e / jnp.sum(e, axis=-1, keepdims=True)

    return pl.pallas_call(
        body,
        out_shape=jax.ShapeDtypeStruct((_BATCH, _OUT), jnp.float32),
    )(pooled_sums, W1, b1.reshape(1, _HIDDEN), W2, b2.reshape(1, _OUT))


def kernel(x, emb, W1, b1, W2, b2):
    tab = _tc_relayout(jnp.transpose(emb))
    pooled_sums = _sc_pooled_sums(x, tab.reshape(_VPAD, _EMBED // 2))
    return _tc_mlp(pooled_sums, W1, b1, W2, b2)


# speedup vs baseline: 3.2646x; 1.0071x over previous
import functools

import jax
import jax.numpy as jnp
from jax import lax
from jax.experimental import pallas as pl
from jax.experimental.pallas import tpu as pltpu
from jax.experimental.pallas import tpu_sc as plsc

_VOCAB = 1000000
_EMBED = 64
_HIDDEN = 128
_OUT = 50
_SEQ = 200
_BATCH = 4096

_NC = 2
_NS = 16
_L = 16
_KV = _EMBED // _L
_NW = _NC * _NS
_BPW = _BATCH // _NW
_SC = 10
_NCHUNK = _SEQ // _SC
_G = 8
_NG = _BPW // _G


_TW = 8192
_H4 = 1 << 18
_VPAD = 4 * _H4
_NBLK_IN = (_VOCAB + _TW - 1) // _TW
_QBLK = _H4 // _TW


def _bf16_bits(x):
    u = lax.bitcast_convert_type(x, jnp.uint32)
    return (u + jnp.uint32(0x7FFF) + ((u >> 16) & jnp.uint32(1))) >> 16


def _tc_relayout(embT):
    def body(a_ref, b_ref, c_ref, d_ref, o_ref):
        ws = []
        for ref in (a_ref, b_ref, c_ref, d_ref):
            wa = _bf16_bits(ref[0:32, :])
            wb = _bf16_bits(ref[32:64, :])
            ws.append(lax.bitcast_convert_type(wa | (wb << 16), jnp.float32))
        o_ref[...] = jnp.transpose(jnp.concatenate(ws, axis=0), (1, 0))

    def qmap(q):
        return lambda i: (0, jnp.minimum(i + q * _QBLK, _NBLK_IN - 1))

    return pl.pallas_call(
        body,
        grid=(_QBLK,),
        in_specs=[pl.BlockSpec((_EMBED, _TW), qmap(q)) for q in range(4)],
        out_specs=pl.BlockSpec((_TW, 2 * _EMBED), lambda i: (i, 0)),
        out_shape=jax.ShapeDtypeStruct((_H4, 2 * _EMBED), jnp.float32),
    )(embT, embT, embT, embT)


def _sc_pooled_sums(x, tab_lin):
    mesh = plsc.VectorSubcoreMesh(
        core_axis_name="c", subcore_axis_name="s",
        num_cores=_NC, num_subcores=_NS)

    @functools.partial(
        pl.kernel,
        out_type=jax.ShapeDtypeStruct((_BATCH, _EMBED), jnp.float32),
        mesh=mesh,
        scratch_types=[
            pltpu.VMEM((_SEQ, _BPW), jnp.int32),
            pltpu.VMEM((2, _SC, _BPW, _EMBED // 2), jnp.float32),
            pltpu.VMEM((_BPW, _EMBED), jnp.float32),
            pltpu.SemaphoreType.DMA,
            pltpu.SemaphoreType.DMA,
        ],
        compiler_params=pltpu.CompilerParams(use_tc_tiling_on_sc=False,
                                             needs_layout_passes=False),
    )
    def body(x_hbm, emb_hbm, out_hbm, idx_v, gbuf, acc_v, sem0, sem1):
        wid = lax.axis_index("s") * _NC + lax.axis_index("c")
        base = wid * _BPW
        pltpu.sync_copy(x_hbm.at[:, pl.ds(base, _BPW)], idx_v)
        sems = (sem0, sem1)

        def remap_body(r, carry):
            for k in range(_BPW // _L):
                w = idx_v[r, pl.ds(k * _L, _L)]
                w2 = ((w & jnp.int32(_H4 - 1)) << 2) | (w >> 18)
                idx_v[r, pl.ds(k * _L, _L)] = w2
            return carry

        lax.fori_loop(0, _SEQ, remap_body, jnp.int32(0))

        def fire(chunk, slot):
            for s in range(_SC):
                pltpu.async_copy(
                    emb_hbm.at[idx_v.at[chunk * _SC + s]],
                    gbuf.at[slot, s], sems[slot])

        def drain(slot):
            for s in range(_SC):
                pltpu.make_async_copy(
                    emb_hbm.at[idx_v.at[0]], gbuf.at[slot, s], sems[slot]).wait()

        def zero_body(r, carry):
            z = jnp.zeros((_L,), jnp.float32)
            for k in range(_KV):
                acc_v[r, pl.ds(k * _L, _L)] = z
            return carry

        lax.fori_loop(0, _BPW, zero_body, jnp.int32(0))

        def process(slot):
            def group_body(g, carry):
                b0 = g * _G
                accs = [[acc_v[b0 + i, pl.ds(k * _L, _L)] for k in range(_KV)]
                        for i in range(_G)]
                for s in range(_SC):
                    for i in range(_G):
                        c0 = plsc.bitcast(gbuf[slot, s, b0 + i, pl.ds(0, _L)],
                                          jnp.bfloat16)
                        c1 = plsc.bitcast(gbuf[slot, s, b0 + i, pl.ds(_L, _L)],
                                          jnp.bfloat16)
                        e0, e2 = plsc.unpack(c0, format=plsc.PackFormat.INTERLEAVED)
                        e1, e3 = plsc.unpack(c1, format=plsc.PackFormat.INTERLEAVED)
                        for k, a in enumerate((e0, e1, e2, e3)):
                            accs[i][k] = accs[i][k] + a
                for i in range(_G):
                    for k in range(_KV):
                        acc_v[b0 + i, pl.ds(k * _L, _L)] = accs[i][k]
                return carry

            lax.fori_loop(0, _NG, group_body, jnp.int32(0))

        fire(0, 0)

        def pair_body(p, carry):
            fire(2 * p + 1, 1)
            drain(0)
            process(0)

            @pl.when(2 * p + 2 < _NCHUNK)
            def _():
                fire(2 * p + 2, 0)

            drain(1)
            process(1)
            return carry

        lax.fori_loop(0, _NCHUNK // 2, pair_body, jnp.int32(0))
        if _NCHUNK % 2:
            drain(0)
            process(0)
        pltpu.sync_copy(acc_v, out_hbm.at[pl.ds(base, _BPW)])

    return body(x, tab_lin)


def _tc_mlp(pooled_sums, W1, b1, W2, b2):
    def body(p_ref, w1_ref, b1_ref, w2_ref, b2_ref, o_ref):
        p = p_ref[...] * (1.0 / _SEQ)
        h = jnp.dot(p, w1_ref[...], preferred_element_type=jnp.float32) + b1_ref[...]
        z = jnp.dot(h, w2_ref[...], preferred_element_type=jnp.float32) + b2_ref[...]
        z = z - jnp.max(z, axis=-1, keepdims=True)
        e = jnp.exp(z)
        o_ref[...] = e / jnp.sum(e, axis=-1, keepdims=True)

    return pl.pallas_call(
        body,
        out_shape=jax.ShapeDtypeStruct((_BATCH, _OUT), jnp.float32),
    )(pooled_sums, W1, b1.reshape(1, _HIDDEN), W2, b2.reshape(1, _OUT))


def kernel(x, emb, W1, b1, W2, b2):
    tab = _tc_relayout(jnp.transpose(emb))
    pooled_sums = _sc_pooled_sums(x, tab.reshape(_VPAD, _EMBED // 2))
    return _tc_mlp(pooled_sums, W1, b1, W2, b2)


# speedup vs baseline: 3.3144x; 1.0153x over previous
import functools

import jax
import jax.numpy as jnp
from jax import lax
from jax.experimental import pallas as pl
from jax.experimental.pallas import tpu as pltpu
from jax.experimental.pallas import tpu_sc as plsc

_VOCAB = 1000000
_EMBED = 64
_HIDDEN = 128
_OUT = 50
_SEQ = 200
_BATCH = 4096

_NC = 2
_NS = 16
_L = 16
_KV = _EMBED // _L
_NW = _NC * _NS
_BPW = _BATCH // _NW
_SC = 10
_NCHUNK = _SEQ // _SC
_G = 8
_NG = _BPW // _G


_TW = 16384
_H4 = 1 << 18
_VPAD = 4 * _H4
_NBLK_IN = (_VOCAB + _TW - 1) // _TW
_QBLK = _H4 // _TW


def _bf16_bits(x):
    u = lax.bitcast_convert_type(x, jnp.uint32)
    return (u + jnp.uint32(0x7FFF) + ((u >> 16) & jnp.uint32(1))) >> 16


def _tc_relayout(embT):
    def body(a_ref, b_ref, c_ref, d_ref, o_ref):
        ws = []
        for ref in (a_ref, b_ref, c_ref, d_ref):
            wa = _bf16_bits(ref[0:32, :])
            wb = _bf16_bits(ref[32:64, :])
            ws.append(lax.bitcast_convert_type(wa | (wb << 16), jnp.float32))
        o_ref[...] = jnp.transpose(jnp.concatenate(ws, axis=0), (1, 0))

    def qmap(q):
        return lambda i: (0, jnp.minimum(i + q * _QBLK, _NBLK_IN - 1))

    return pl.pallas_call(
        body,
        grid=(_QBLK,),
        in_specs=[pl.BlockSpec((_EMBED, _TW), qmap(q)) for q in range(4)],
        out_specs=pl.BlockSpec((_TW, 2 * _EMBED), lambda i: (i, 0)),
        out_shape=jax.ShapeDtypeStruct((_H4, 2 * _EMBED), jnp.float32),
    )(embT, embT, embT, embT)


def _sc_pooled_sums(x, tab_lin):
    mesh = plsc.VectorSubcoreMesh(
        core_axis_name="c", subcore_axis_name="s",
        num_cores=_NC, num_subcores=_NS)

    @functools.partial(
        pl.kernel,
        out_type=jax.ShapeDtypeStruct((_BATCH, _EMBED), jnp.float32),
        mesh=mesh,
        scratch_types=[
            pltpu.VMEM((_SEQ, _BPW), jnp.int32),
            pltpu.VMEM((2, _SC, _BPW, _EMBED // 2), jnp.float32),
            pltpu.VMEM((_BPW, _EMBED), jnp.float32),
            pltpu.SemaphoreType.DMA,
            pltpu.SemaphoreType.DMA,
        ],
        compiler_params=pltpu.CompilerParams(use_tc_tiling_on_sc=False,
                                             needs_layout_passes=False),
    )
    def body(x_hbm, emb_hbm, out_hbm, idx_v, gbuf, acc_v, sem0, sem1):
        wid = lax.axis_index("s") * _NC + lax.axis_index("c")
        base = wid * _BPW
        pltpu.sync_copy(x_hbm.at[:, pl.ds(base, _BPW)], idx_v)
        sems = (sem0, sem1)

        def remap_body(r, carry):
            for k in range(_BPW // _L):
                w = idx_v[r, pl.ds(k * _L, _L)]
                w2 = ((w & jnp.int32(_H4 - 1)) << 2) | (w >> 18)
                idx_v[r, pl.ds(k * _L, _L)] = w2
            return carry

        lax.fori_loop(0, _SEQ, remap_body, jnp.int32(0))

        def fire(chunk, slot):
            for s in range(_SC):
                pltpu.async_copy(
                    emb_hbm.at[idx_v.at[chunk * _SC + s]],
                    gbuf.at[slot, s], sems[slot])

        def drain(slot):
            for s in range(_SC):
                pltpu.make_async_copy(
                    emb_hbm.at[idx_v.at[0]], gbuf.at[slot, s], sems[slot]).wait()

        def zero_body(r, carry):
            z = jnp.zeros((_L,), jnp.float32)
            for k in range(_KV):
                acc_v[r, pl.ds(k * _L, _L)] = z
            return carry

        lax.fori_loop(0, _BPW, zero_body, jnp.int32(0))

        def process(slot):
            def group_body(g, carry):
                b0 = g * _G
                accs = [[acc_v[b0 + i, pl.ds(k * _L, _L)] for k in range(_KV)]
                        for i in range(_G)]
                for s in range(_SC):
                    for i in range(_G):
                        c0 = plsc.bitcast(gbuf[slot, s, b0 + i, pl.ds(0, _L)],
                                          jnp.bfloat16)
                        c1 = plsc.bitcast(gbuf[slot, s, b0 + i, pl.ds(_L, _L)],
                                          jnp.bfloat16)
                        e0, e2 = plsc.unpack(c0, format=plsc.PackFormat.INTERLEAVED)
                        e1, e3 = plsc.unpack(c1, format=plsc.PackFormat.INTERLEAVED)
                        for k, a in enumerate((e0, e1, e2, e3)):
                            accs[i][k] = accs[i][k] + a
                for i in range(_G):
                    for k in range(_KV):
                        acc_v[b0 + i, pl.ds(k * _L, _L)] = accs[i][k]
                return carry

            lax.fori_loop(0, _NG, group_body, jnp.int32(0))

        fire(0, 0)

        def pair_body(p, carry):
            fire(2 * p + 1, 1)
            drain(0)
            process(0)

            @pl.when(2 * p + 2 < _NCHUNK)
            def _():
                fire(2 * p + 2, 0)

            drain(1)
            process(1)
            return carry

        lax.fori_loop(0, _NCHUNK // 2, pair_body, jnp.int32(0))
        if _NCHUNK % 2:
            drain(0)
            process(0)
        pltpu.sync_copy(acc_v, out_hbm.at[pl.ds(base, _BPW)])

    return body(x, tab_lin)


def _tc_mlp(pooled_sums, W1, b1, W2, b2):
    def body(p_ref, w1_ref, b1_ref, w2_ref, b2_ref, o_ref):
        p = p_ref[...] * (1.0 / _SEQ)
        h = jnp.dot(p, w1_ref[...], preferred_element_type=jnp.float32) + b1_ref[...]
        z = jnp.dot(h, w2_ref[...], preferred_element_type=jnp.float32) + b2_ref[...]
        z = z - jnp.max(z, axis=-1, keepdims=True)
        e = jnp.exp(z)
        o_ref[...] = e / jnp.sum(e, axis=-1, keepdims=True)

    return pl.pallas_call(
        body,
        out_shape=jax.ShapeDtypeStruct((_BATCH, _OUT), jnp.float32),
    )(pooled_sums, W1, b1.reshape(1, _HIDDEN), W2, b2.reshape(1, _OUT))


def kernel(x, emb, W1, b1, W2, b2):
    tab = _tc_relayout(jnp.transpose(emb))
    pooled_sums = _sc_pooled_sums(x, tab.reshape(_VPAD, _EMBED // 2))
    return _tc_mlp(pooled_sums, W1, b1, W2, b2)


# speedup vs baseline: 3.3463x; 1.0096x over previous
import functools

import jax
import jax.numpy as jnp
from jax import lax
from jax.experimental import pallas as pl
from jax.experimental.pallas import tpu as pltpu
from jax.experimental.pallas import tpu_sc as plsc

_VOCAB = 1000000
_EMBED = 64
_HIDDEN = 128
_OUT = 50
_SEQ = 200
_BATCH = 4096

_NC = 2
_NS = 16
_L = 16
_KV = _EMBED // _L
_NW = _NC * _NS
_BPW = _BATCH // _NW
_SC = 10
_NCHUNK = _SEQ // _SC
_G = 8
_NG = _BPW // _G


_TW = 16384
_H4 = 1 << 18
_VPAD = 4 * _H4
_NBLK_IN = (_VOCAB + _TW - 1) // _TW
_QBLK = _H4 // _TW


def _bf16_bits(x):
    u = lax.bitcast_convert_type(x, jnp.uint32)
    return (u + jnp.uint32(0x7FFF) + ((u >> 16) & jnp.uint32(1))) >> 16


def _tc_relayout(embT):
    def body(a_ref, b_ref, c_ref, d_ref, o_ref):
        ws = []
        for ref in (a_ref, b_ref, c_ref, d_ref):
            wa = _bf16_bits(ref[0:32, :])
            wb = _bf16_bits(ref[32:64, :])
            ws.append(lax.bitcast_convert_type(wa | (wb << 16), jnp.float32))
        o_ref[...] = jnp.transpose(jnp.concatenate(ws, axis=0), (1, 0))

    def qmap(q):
        return lambda i: (0, jnp.minimum(i + q * _QBLK, _NBLK_IN - 1))

    return pl.pallas_call(
        body,
        grid=(_QBLK,),
        in_specs=[pl.BlockSpec((_EMBED, _TW), qmap(q)) for q in range(4)],
        out_specs=pl.BlockSpec((_TW, 2 * _EMBED), lambda i: (i, 0)),
        out_shape=jax.ShapeDtypeStruct((_H4, 2 * _EMBED), jnp.float32),
    )(embT, embT, embT, embT)


def _sc_pooled_sums(x, tab_lin):
    mesh = plsc.VectorSubcoreMesh(
        core_axis_name="c", subcore_axis_name="s",
        num_cores=_NC, num_subcores=_NS)

    @functools.partial(
        pl.kernel,
        out_type=jax.ShapeDtypeStruct((_BATCH, _EMBED), jnp.float32),
        mesh=mesh,
        scratch_types=[
            pltpu.VMEM((_SEQ, _BPW), jnp.int32),
            pltpu.VMEM((2, _SC, _BPW, _EMBED // 2), jnp.float32),
            pltpu.VMEM((_BPW, _EMBED), jnp.float32),
            pltpu.SemaphoreType.DMA,
            pltpu.SemaphoreType.DMA,
        ],
        compiler_params=pltpu.CompilerParams(use_tc_tiling_on_sc=False,
                                             needs_layout_passes=False),
    )
    def body(x_hbm, emb_hbm, out_hbm, idx_v, gbuf, acc_v, sem0, sem1):
        wid = lax.axis_index("s") * _NC + lax.axis_index("c")
        base = wid * _BPW
        sems = (sem0, sem1)

        def remap_body(r, carry):
            for k in range(_BPW // _L):
                w = idx_v[r, pl.ds(k * _L, _L)]
                w2 = ((w & jnp.int32(_H4 - 1)) << 2) | (w >> 18)
                idx_v[r, pl.ds(k * _L, _L)] = w2
            return carry

        def fire(chunk, slot):
            for s in range(_SC):
                pltpu.async_copy(
                    emb_hbm.at[idx_v.at[chunk * _SC + s]],
                    gbuf.at[slot, s], sems[slot])

        def drain(slot):
            for s in range(_SC):
                pltpu.make_async_copy(
                    emb_hbm.at[idx_v.at[0]], gbuf.at[slot, s], sems[slot]).wait()

        def zero_body(r, carry):
            z = jnp.zeros((_L,), jnp.float32)
            for k in range(_KV):
                acc_v[r, pl.ds(k * _L, _L)] = z
            return carry

        lax.fori_loop(0, _BPW, zero_body, jnp.int32(0))

        def process(slot):
            def group_body(g, carry):
                b0 = g * _G
                accs = [[acc_v[b0 + i, pl.ds(k * _L, _L)] for k in range(_KV)]
                        for i in range(_G)]
                for s in range(_SC):
                    for i in range(_G):
                        c0 = plsc.bitcast(gbuf[slot, s, b0 + i, pl.ds(0, _L)],
                                          jnp.bfloat16)
                        c1 = plsc.bitcast(gbuf[slot, s, b0 + i, pl.ds(_L, _L)],
                                          jnp.bfloat16)
                        e0, e2 = plsc.unpack(c0, format=plsc.PackFormat.INTERLEAVED)
                        e1, e3 = plsc.unpack(c1, format=plsc.PackFormat.INTERLEAVED)
                        for k, a in enumerate((e0, e1, e2, e3)):
                            accs[i][k] = accs[i][k] + a
                for i in range(_G):
                    for k in range(_KV):
                        acc_v[b0 + i, pl.ds(k * _L, _L)] = accs[i][k]
                return carry

            lax.fori_loop(0, _NG, group_body, jnp.int32(0))

        pltpu.sync_copy(x_hbm.at[pl.ds(0, _SC), pl.ds(base, _BPW)],
                        idx_v.at[pl.ds(0, _SC)])
        lax.fori_loop(0, _SC, remap_body, jnp.int32(0))
        fire(0, 0)
        pltpu.sync_copy(x_hbm.at[pl.ds(_SC, _SEQ - _SC), pl.ds(base, _BPW)],
                        idx_v.at[pl.ds(_SC, _SEQ - _SC)])
        lax.fori_loop(_SC, _SEQ, remap_body, jnp.int32(0))

        def pair_body(p, carry):
            fire(2 * p + 1, 1)
            drain(0)
            process(0)

            @pl.when(2 * p + 2 < _NCHUNK)
            def _():
                fire(2 * p + 2, 0)

            drain(1)
            process(1)
            return carry

        lax.fori_loop(0, _NCHUNK // 2, pair_body, jnp.int32(0))
        if _NCHUNK % 2:
            drain(0)
            process(0)
        pltpu.sync_copy(acc_v, out_hbm.at[pl.ds(base, _BPW)])

    return body(x, tab_lin)


def _tc_mlp(pooled_sums, W1, b1, W2, b2):
    def body(p_ref, w1_ref, b1_ref, w2_ref, b2_ref, o_ref):
        p = p_ref[...] * (1.0 / _SEQ)
        h = jnp.dot(p, w1_ref[...], preferred_element_type=jnp.float32) + b1_ref[...]
        z = jnp.dot(h, w2_ref[...], preferred_element_type=jnp.float32) + b2_ref[...]
        z = z - jnp.max(z, axis=-1, keepdims=True)
        e = jnp.exp(z)
        o_ref[...] = e / jnp.sum(e, axis=-1, keepdims=True)

    return pl.pallas_call(
        body,
        out_shape=jax.ShapeDtypeStruct((_BATCH, _OUT), jnp.float32),
    )(pooled_sums, W1, b1.reshape(1, _HIDDEN), W2, b2.reshape(1, _OUT))


def kernel(x, emb, W1, b1, W2, b2):
    tab = _tc_relayout(jnp.transpose(emb))
    pooled_sums = _sc_pooled_sums(x, tab.reshape(_VPAD, _EMBED // 2))
    return _tc_mlp(pooled_sums, W1, b1, W2, b2)
